# baseline (speedup 1.0000x reference)
import functools

import jax
import jax.numpy as jnp
from jax import lax
from jax.experimental import pallas as pl
from jax.experimental.pallas import tpu as pltpu
from jax.experimental.pallas import tpu_sc as plsc


def _make_gather(B, V, Da, Db):
    info = plsc.get_sparse_core_info()
    NW = info.num_cores * info.num_subcores
    assert B % (8 * NW) == 0
    b_per_w = B // NW
    C = 128
    NCH = b_per_w // C
    assert NCH * C == b_per_w
    ra = 128 // Da
    rb = 128 // Db
    sa = ra.bit_length() - 1
    sb = rb.bit_length() - 1
    mesh = plsc.VectorSubcoreMesh(core_axis_name="c", subcore_axis_name="s")

    @functools.partial(
        pl.kernel,
        mesh=mesh,
        out_type=(
            jax.ShapeDtypeStruct((B, Da), jnp.float32),
            jax.ShapeDtypeStruct((B, Db), jnp.float32),
        ),
        scratch_types=[
            pltpu.VMEM((b_per_w,), jnp.int32),
            pltpu.VMEM((C,), jnp.int32),
            pltpu.VMEM((C,), jnp.int32),
            pltpu.VMEM((C, 128), jnp.float32),
            pltpu.VMEM((C, 128), jnp.float32),
            pltpu.VMEM((C, Da), jnp.float32),
            pltpu.VMEM((C, Db), jnp.float32),
            pltpu.SemaphoreType.DMA,
            pltpu.SemaphoreType.DMA,
        ],
    )
    def gather_k(idx_hbm, a_hbm, b_hbm, out_a_hbm, out_b_hbm,
                 idx_v, blk_a, blk_b, tiles_a, tiles_b, rows_a, rows_b,
                 sem_a, sem_b):
        wid = lax.axis_index("s") * info.num_cores + lax.axis_index("c")
        base = wid * b_per_w
        pltpu.sync_copy(idx_hbm.at[pl.ds(base, b_per_w)], idx_v)

        def chunk_body(g, carry):
            off = g * C
            for j in range(C // 16):
                vec = idx_v[pl.ds(off + j * 16, 16)]
                blk_a[pl.ds(j * 16, 16)] = lax.shift_right_logical(vec, sa)
                blk_b[pl.ds(j * 16, 16)] = lax.shift_right_logical(vec, sb)
            cp_a = pltpu.async_copy(a_hbm.at[blk_a], tiles_a, sem_a)
            cp_b = pltpu.async_copy(b_hbm.at[blk_b], tiles_b, sem_b)
            cp_a.wait()
            for j in range(C // 16):
                vec = idx_v[pl.ds(off + j * 16, 16)]
                hvec = lax.bitwise_and(vec, ra - 1) * Da
                for k in range(16):
                    i = j * 16 + k
                    h = hvec[k]
                    for m in range(Da // 16):
                        rows_a[i, pl.ds(m * 16, 16)] = (
                            tiles_a[i, pl.ds(h + m * 16, 16)])
            cp_b.wait()
            for j in range(C // 16):
                vec = idx_v[pl.ds(off + j * 16, 16)]
                hvec = lax.bitwise_and(vec, rb - 1) * Db
                for k in range(16):
                    i = j * 16 + k
                    h = hvec[k]
                    rows_b[i, :] = tiles_b[i, pl.ds(h, 16)]
            pltpu.sync_copy(rows_a, out_a_hbm.at[pl.ds(base + off, C)])
            pltpu.sync_copy(rows_b, out_b_hbm.at[pl.ds(base + off, C)])
            return carry

        lax.fori_loop(0, NCH, chunk_body, 0)

    return gather_k


def kernel(indices, values_a, values_b):
    B = indices.shape[0]
    V, Da = values_a.shape
    Db = values_b.shape[1]
    gather_k = _make_gather(B, V, Da, Db)
    va = values_a.reshape(V * Da // 128, 128)
    vb = values_b.reshape(V * Db // 128, 128)
    out_a, out_b = gather_k(indices[:, 0], va, vb)
    return (out_a, out_b)


# speedup vs baseline: 2.1688x; 2.1688x over previous
import functools

import jax
import jax.numpy as jnp
from jax import lax
from jax.experimental import pallas as pl
from jax.experimental.pallas import tpu as pltpu
from jax.experimental.pallas import tpu_sc as plsc


def _make_gather(B, V, Da, Db):
    info = plsc.get_sparse_core_info()
    NW = info.num_cores * info.num_subcores
    assert B % (8 * NW) == 0 and V % 8 == 0
    b_per_w = B // NW
    C = 32
    NCH = b_per_w // C
    assert NCH * C == b_per_w
    mesh = plsc.VectorSubcoreMesh(core_axis_name="c", subcore_axis_name="s")

    @functools.partial(
        pl.kernel,
        mesh=mesh,
        out_type=(
            jax.ShapeDtypeStruct((B, Da), jnp.float32),
            jax.ShapeDtypeStruct((B, Db), jnp.float32),
        ),
        scratch_types=[
            pltpu.VMEM((b_per_w,), jnp.int32),
            pltpu.VMEM((C, 8, Da), jnp.float32),
            pltpu.VMEM((C, 8, Db), jnp.float32),
            pltpu.VMEM((C, Da), jnp.float32),
            pltpu.VMEM((C, Db), jnp.float32),
            pltpu.SemaphoreType.DMA,
            pltpu.SemaphoreType.DMA,
        ],
    )
    def gather_k(idx_hbm, a_hbm, b_hbm, out_a_hbm, out_b_hbm,
                 idx_v, tiles_a, tiles_b, rows_a, rows_b,
                 sem_a, sem_b):
        wid = lax.axis_index("s") * info.num_cores + lax.axis_index("c")
        base = wid * b_per_w
        pltpu.sync_copy(idx_hbm.at[pl.ds(base, b_per_w)], idx_v)

        def chunk_body(g, carry):
            off = g * C
            for j in range(C // 16):
                vec = idx_v[pl.ds(off + j * 16, 16)]
                tvec = lax.shift_right_logical(vec, 3)
                for k in range(16):
                    i = j * 16 + k
                    t = tvec[k]
                    pltpu.async_copy(a_hbm.at[t], tiles_a.at[i], sem_a)
                    pltpu.async_copy(b_hbm.at[t], tiles_b.at[i], sem_b)
            pltpu.make_async_copy(a_hbm.at[pl.ds(0, C)], tiles_a, sem_a).wait()
            pltpu.make_async_copy(b_hbm.at[pl.ds(0, C)], tiles_b, sem_b).wait()
            for j in range(C // 16):
                vec = idx_v[pl.ds(off + j * 16, 16)]
                uvec = lax.bitwise_and(vec, 7)
                for k in range(16):
                    i = j * 16 + k
                    u = uvec[k]
                    for m in range(Da // 16):
                        rows_a[i, pl.ds(m * 16, 16)] = (
                            tiles_a[i, u, pl.ds(m * 16, 16)])
                    for m in range(Db // 16):
                        rows_b[i, pl.ds(m * 16, 16)] = (
                            tiles_b[i, u, pl.ds(m * 16, 16)])
            pltpu.sync_copy(rows_a, out_a_hbm.at[pl.ds(base + off, C)])
            pltpu.sync_copy(rows_b, out_b_hbm.at[pl.ds(base + off, C)])
            return carry

        lax.fori_loop(0, NCH, chunk_body, 0)

    return gather_k


def kernel(indices, values_a, values_b):
    B = indices.shape[0]
    V, Da = values_a.shape
    Db = values_b.shape[1]
    gather_k = _make_gather(B, V, Da, Db)
    va = values_a.reshape(V // 8, 8, Da)
    vb = values_b.reshape(V // 8, 8, Db)
    out_a, out_b = gather_k(indices[:, 0], va, vb)
    return (out_a, out_b)
